# Initial kernel scaffold; baseline (speedup 1.0000x reference)
import functools

import jax
import jax.numpy as jnp
from jax import lax
from jax.experimental import pallas as pl
from jax.experimental.pallas import tpu as pltpu
from jax.experimental.pallas import tpu_sc as plsc

_N = 10000
_E = 320000
_D = 128
_NC = 2
_NS = 16
_NW = _NC * _NS
_EPW = _E // _NW
_C = 80
_NCHUNK = _EPW // _C
_RC = 125
_RPT = _N // _NS
_DPT = 624
_DTAIL = _N - _NS * _DPT

_mesh = plsc.VectorSubcoreMesh(core_axis_name="c", subcore_axis_name="s")
_f32 = jnp.float32


@functools.partial(
    pl.kernel,
    out_type=jax.ShapeDtypeStruct((_NC, _N), _f32),
    mesh=_mesh,
    scratch_types=[
        pltpu.VMEM((_C,), jnp.int32),
        pltpu.VMEM((_C,), _f32),
        pltpu.VMEM((_DPT,), _f32),
        pltpu.VMEM_SHARED((_N,), _f32),
    ],
)
def _deg_kernel(dst_hbm, out_hbm, idx_v, ones_v, buf_v, acc_sh):
    c = lax.axis_index("c")
    s = lax.axis_index("s")
    wid = c * _NS + s

    for j in range(_C // 16):
        ones_v[pl.ds(j * 16, 16)] = jnp.ones((16,), _f32)
    for j in range(_DPT // 16):
        buf_v[pl.ds(j * 16, 16)] = jnp.zeros((16,), _f32)
    pltpu.sync_copy(buf_v, acc_sh.at[pl.ds(s * _DPT, _DPT)])

    @pl.when(s == _NS - 1)
    def _():
        pltpu.sync_copy(buf_v.at[pl.ds(0, _DTAIL)],
                        acc_sh.at[pl.ds(_NS * _DPT, _DTAIL)])

    plsc.subcore_barrier()

    def body(i, carry):
        pltpu.sync_copy(dst_hbm.at[pl.ds(wid * _EPW + i * _C, _C)], idx_v)
        pltpu.sync_copy(ones_v, acc_sh.at[idx_v], add=True)
        return carry

    lax.fori_loop(0, _NCHUNK, body, 0)
    plsc.subcore_barrier()

    pltpu.sync_copy(acc_sh.at[pl.ds(s * _DPT, _DPT)], buf_v)
    pltpu.sync_copy(buf_v, out_hbm.at[c, pl.ds(s * _DPT, _DPT)])

    @pl.when(s == _NS - 1)
    def _():
        pltpu.sync_copy(acc_sh.at[pl.ds(_NS * _DPT, _DTAIL)],
                        buf_v.at[pl.ds(0, _DTAIL)])
        pltpu.sync_copy(buf_v.at[pl.ds(0, _DTAIL)],
                        out_hbm.at[c, pl.ds(_NS * _DPT, _DTAIL)])


@functools.partial(
    pl.kernel,
    out_type=jax.ShapeDtypeStruct((_NC, _N, _D), _f32),
    mesh=_mesh,
    scratch_types=[
        pltpu.VMEM((_C,), jnp.int32),
        pltpu.VMEM((_C,), jnp.int32),
        pltpu.VMEM((_C, _D), _f32),
        pltpu.VMEM((_RC, _D), _f32),
        pltpu.VMEM_SHARED((_N, _D), _f32),
        pltpu.SemaphoreType.DMA,
    ],
)
def _hop_kernel(g_hbm, src_hbm, dst_hbm, out_hbm,
                srcv, dstv, rows, buf, acc_sh, sem):
    c = lax.axis_index("c")
    s = lax.axis_index("s")
    wid = c * _NS + s

    def zrow(r, carry):
        for j in range(_D // 16):
            buf[r, pl.ds(j * 16, 16)] = jnp.zeros((16,), _f32)
        return carry

    lax.fori_loop(0, _RC, zrow, 0)
    for k in range(_RPT // _RC):
        pltpu.sync_copy(buf, acc_sh.at[pl.ds(s * _RPT + k * _RC, _RC)])

    plsc.subcore_barrier()

    def body(i, carry):
        off = wid * _EPW + i * _C
        pltpu.sync_copy(src_hbm.at[pl.ds(off, _C)], srcv)
        pltpu.sync_copy(dst_hbm.at[pl.ds(off, _C)], dstv)
        pltpu.async_copy(g_hbm.at[srcv], rows, sem).wait()
        pltpu.sync_copy(rows, acc_sh.at[dstv], add=True)
        return carry

    lax.fori_loop(0, _NCHUNK, body, 0)
    plsc.subcore_barrier()

    for k in range(_RPT // _RC):
        r0 = s * _RPT + k * _RC
        pltpu.sync_copy(acc_sh.at[pl.ds(r0, _RC)], buf)
        pltpu.sync_copy(buf, out_hbm.at[c, pl.ds(r0, _RC)])


_BR = 2000


def _norm_mul_body(degp_ref, feat_ref, norm_ref, g1_ref):
    d = degp_ref[0] + degp_ref[1]
    nv = lax.rsqrt(jnp.maximum(d, 1.0))
    norm_ref[...] = nv
    g1_ref[...] = feat_ref[...] * nv


def _mid_body(p_ref, norm_ref, g2_ref):
    nv = norm_ref[...]
    g2_ref[...] = (p_ref[0] + p_ref[1]) * (nv * nv)


def _fc_body(p_ref, norm_ref, w_ref, b_ref, out_ref):
    h = (p_ref[0] + p_ref[1]) * norm_ref[...]
    out_ref[...] = (
        jnp.dot(h, w_ref[...], preferred_element_type=_f32) + b_ref[...]
    )


_norm_call = pl.pallas_call(
    _norm_mul_body,
    grid=(_N // _BR,),
    in_specs=[
        pl.BlockSpec((_NC, _BR, 1), lambda i: (0, i, 0)),
        pl.BlockSpec((_BR, _D), lambda i: (i, 0)),
    ],
    out_specs=[
        pl.BlockSpec((_BR, 1), lambda i: (i, 0)),
        pl.BlockSpec((_BR, _D), lambda i: (i, 0)),
    ],
    out_shape=[
        jax.ShapeDtypeStruct((_N, 1), _f32),
        jax.ShapeDtypeStruct((_N, _D), _f32),
    ],
)

_mid_call = pl.pallas_call(
    _mid_body,
    grid=(_N // _BR,),
    in_specs=[
        pl.BlockSpec((_NC, _BR, _D), lambda i: (0, i, 0)),
        pl.BlockSpec((_BR, 1), lambda i: (i, 0)),
    ],
    out_specs=pl.BlockSpec((_BR, _D), lambda i: (i, 0)),
    out_shape=jax.ShapeDtypeStruct((_N, _D), _f32),
)

_fc_call = pl.pallas_call(
    _fc_body,
    grid=(_N // _BR,),
    in_specs=[
        pl.BlockSpec((_NC, _BR, _D), lambda i: (0, i, 0)),
        pl.BlockSpec((_BR, 1), lambda i: (i, 0)),
        pl.BlockSpec((_D, _D), lambda i: (0, 0)),
        pl.BlockSpec((1, _D), lambda i: (0, 0)),
    ],
    out_specs=pl.BlockSpec((_BR, _D), lambda i: (i, 0)),
    out_shape=jax.ShapeDtypeStruct((_N, _D), _f32),
)


def kernel(feat, edge_index, W, b):
    src = edge_index[0]
    dst = edge_index[1]
    degp = _deg_kernel(dst)
    norm, g1 = _norm_call(degp.reshape(_NC, _N, 1), feat)
    p1 = _hop_kernel(g1, src, dst)
    g2 = _mid_call(p1, norm)
    p2 = _hop_kernel(g2, src, dst)
    out = _fc_call(p2, norm, W, b.reshape(1, _D))
    return out


# speedup vs baseline: 4.2096x; 4.2096x over previous
import functools

import jax
import jax.numpy as jnp
from jax import lax
from jax.experimental import pallas as pl
from jax.experimental.pallas import tpu as pltpu
from jax.experimental.pallas import tpu_sc as plsc

_N = 10000
_E = 320000
_D = 128
_NC = 2
_NS = 16
_NW = _NC * _NS
_EPW = _E // _NW
_C = 80
_NCHUNK = _EPW // _C
_RC = 104
_RPT = 624
_RTAIL = _N - _NS * _RPT
_DPT = 624
_DTAIL = _N - _NS * _DPT

_mesh = plsc.VectorSubcoreMesh(core_axis_name="c", subcore_axis_name="s")
_f32 = jnp.float32


@functools.partial(
    pl.kernel,
    out_type=jax.ShapeDtypeStruct((_NC * _N,), _f32),
    mesh=_mesh,
    scratch_types=[
        pltpu.VMEM((_C,), jnp.int32),
        pltpu.VMEM((_C,), _f32),
        pltpu.VMEM((_DPT,), _f32),
        pltpu.VMEM_SHARED((_N,), _f32),
    ],
)
def _deg_kernel(dst_hbm, out_hbm, idx_v, ones_v, buf_v, acc_sh):
    c = lax.axis_index("c")
    s = lax.axis_index("s")
    wid = c * _NS + s

    for j in range(_C // 16):
        ones_v[pl.ds(j * 16, 16)] = jnp.ones((16,), _f32)
    for j in range(_DPT // 16):
        buf_v[pl.ds(j * 16, 16)] = jnp.zeros((16,), _f32)
    pltpu.sync_copy(buf_v, acc_sh.at[pl.ds(s * _DPT, _DPT)])

    @pl.when(s == _NS - 1)
    def _():
        pltpu.sync_copy(buf_v.at[pl.ds(0, _DTAIL)],
                        acc_sh.at[pl.ds(_NS * _DPT, _DTAIL)])

    plsc.subcore_barrier()

    def body(i, carry):
        pltpu.sync_copy(dst_hbm.at[pl.ds(wid * _EPW + i * _C, _C)], idx_v)
        pltpu.sync_copy(ones_v, acc_sh.at[idx_v], add=True)
        return carry

    lax.fori_loop(0, _NCHUNK, body, 0)
    plsc.subcore_barrier()

    pltpu.sync_copy(acc_sh.at[pl.ds(s * _DPT, _DPT)], buf_v)
    pltpu.sync_copy(buf_v, out_hbm.at[pl.ds(c * _N + s * _DPT, _DPT)])

    @pl.when(s == _NS - 1)
    def _():
        pltpu.sync_copy(acc_sh.at[pl.ds(_NS * _DPT, _DTAIL)],
                        buf_v.at[pl.ds(0, _DTAIL)])
        pltpu.sync_copy(buf_v.at[pl.ds(0, _DTAIL)],
                        out_hbm.at[pl.ds(c * _N + _NS * _DPT, _DTAIL)])


@functools.partial(
    pl.kernel,
    out_type=jax.ShapeDtypeStruct((_NC, _N, _D), _f32),
    mesh=_mesh,
    scratch_types=[
        pltpu.VMEM((_C,), jnp.int32),
        pltpu.VMEM((_C,), jnp.int32),
        pltpu.VMEM((_C, _D), _f32),
        pltpu.VMEM((_RC, _D), _f32),
        pltpu.VMEM_SHARED((_N, _D), _f32),
        pltpu.SemaphoreType.DMA,
    ],
)
def _hop_kernel(g_hbm, src_hbm, dst_hbm, out_hbm,
                srcv, dstv, rows, buf, acc_sh, sem):
    c = lax.axis_index("c")
    s = lax.axis_index("s")
    wid = c * _NS + s

    def zrow(r, carry):
        for j in range(_D // 16):
            buf[r, pl.ds(j * 16, 16)] = jnp.zeros((16,), _f32)
        return carry

    lax.fori_loop(0, _RC, zrow, 0)
    for k in range(_RPT // _RC):
        pltpu.sync_copy(buf, acc_sh.at[pl.ds(s * _RPT + k * _RC, _RC)])

    @pl.when(s == _NS - 1)
    def _():
        pltpu.sync_copy(buf.at[pl.ds(0, _RTAIL)],
                        acc_sh.at[pl.ds(_NS * _RPT, _RTAIL)])

    plsc.subcore_barrier()

    def body(i, carry):
        off = wid * _EPW + i * _C
        pltpu.sync_copy(src_hbm.at[pl.ds(off, _C)], srcv)
        pltpu.sync_copy(dst_hbm.at[pl.ds(off, _C)], dstv)
        pltpu.async_copy(g_hbm.at[srcv], rows, sem).wait()
        pltpu.sync_copy(rows, acc_sh.at[dstv], add=True)
        return carry

    lax.fori_loop(0, _NCHUNK, body, 0)
    plsc.subcore_barrier()

    for k in range(_RPT // _RC):
        r0 = s * _RPT + k * _RC
        pltpu.sync_copy(acc_sh.at[pl.ds(r0, _RC)], buf)
        pltpu.sync_copy(buf, out_hbm.at[c, pl.ds(r0, _RC)])

    @pl.when(s == _NS - 1)
    def _():
        pltpu.sync_copy(acc_sh.at[pl.ds(_NS * _RPT, _RTAIL)],
                        buf.at[pl.ds(0, _RTAIL)])
        pltpu.sync_copy(buf.at[pl.ds(0, _RTAIL)],
                        out_hbm.at[c, pl.ds(_NS * _RPT, _RTAIL)])


_BR = 2000


def _norm_mul_body(degp_ref, feat_ref, norm_ref, g1_ref):
    d = degp_ref[0] + degp_ref[1]
    nv = lax.rsqrt(jnp.maximum(d, 1.0))
    norm_ref[...] = nv
    g1_ref[...] = feat_ref[...] * nv


def _mid_body(p_ref, norm_ref, g2_ref):
    nv = norm_ref[...]
    g2_ref[...] = (p_ref[0] + p_ref[1]) * (nv * nv)


def _fc_body(p_ref, norm_ref, w_ref, b_ref, out_ref):
    h = (p_ref[0] + p_ref[1]) * norm_ref[...]
    out_ref[...] = (
        jnp.dot(h, w_ref[...], preferred_element_type=_f32) + b_ref[...]
    )


_norm_call = pl.pallas_call(
    _norm_mul_body,
    grid=(_N // _BR,),
    in_specs=[
        pl.BlockSpec((_NC, _BR, 1), lambda i: (0, i, 0)),
        pl.BlockSpec((_BR, _D), lambda i: (i, 0)),
    ],
    out_specs=[
        pl.BlockSpec((_BR, 1), lambda i: (i, 0)),
        pl.BlockSpec((_BR, _D), lambda i: (i, 0)),
    ],
    out_shape=[
        jax.ShapeDtypeStruct((_N, 1), _f32),
        jax.ShapeDtypeStruct((_N, _D), _f32),
    ],
)

_mid_call = pl.pallas_call(
    _mid_body,
    grid=(_N // _BR,),
    in_specs=[
        pl.BlockSpec((_NC, _BR, _D), lambda i: (0, i, 0)),
        pl.BlockSpec((_BR, 1), lambda i: (i, 0)),
    ],
    out_specs=pl.BlockSpec((_BR, _D), lambda i: (i, 0)),
    out_shape=jax.ShapeDtypeStruct((_N, _D), _f32),
)

_fc_call = pl.pallas_call(
    _fc_body,
    grid=(_N // _BR,),
    in_specs=[
        pl.BlockSpec((_NC, _BR, _D), lambda i: (0, i, 0)),
        pl.BlockSpec((_BR, 1), lambda i: (i, 0)),
        pl.BlockSpec((_D, _D), lambda i: (0, 0)),
        pl.BlockSpec((1, _D), lambda i: (0, 0)),
    ],
    out_specs=pl.BlockSpec((_BR, _D), lambda i: (i, 0)),
    out_shape=jax.ShapeDtypeStruct((_N, _D), _f32),
)


def kernel(feat, edge_index, W, b):
    src = edge_index[0]
    dst = edge_index[1]
    degp = _deg_kernel(dst)
    norm, g1 = _norm_call(degp.reshape(_NC, _N, 1), feat)
    p1 = _hop_kernel(g1, src, dst)
    g2 = _mid_call(p1, norm)
    p2 = _hop_kernel(g2, src, dst)
    out = _fc_call(p2, norm, W, b.reshape(1, _D))
    return out


# speedup vs baseline: 8.0912x; 1.9221x over previous
import functools

import jax
import jax.numpy as jnp
from jax import lax
from jax.experimental import pallas as pl
from jax.experimental.pallas import tpu as pltpu
from jax.experimental.pallas import tpu_sc as plsc

_N = 10000
_E = 320000
_D = 128
_NC = 2
_NS = 16
_NW = _NC * _NS
_EPW = _E // _NW
_C = 128
_M = _EPW // _C
_CT = _EPW - _M * _C
_RC = 104
_RPT = 624
_RTAIL = _N - _NS * _RPT
_DPT = 624
_DTAIL = _N - _NS * _DPT

_mesh = plsc.VectorSubcoreMesh(core_axis_name="c", subcore_axis_name="s")
_f32 = jnp.float32


@functools.partial(
    pl.kernel,
    out_type=jax.ShapeDtypeStruct((_NC * _N,), _f32),
    mesh=_mesh,
    scratch_types=[
        pltpu.VMEM((_C,), jnp.int32),
        pltpu.VMEM((_C,), jnp.int32),
        pltpu.VMEM((_CT,), jnp.int32),
        pltpu.VMEM((_C,), _f32),
        pltpu.VMEM((_DPT,), _f32),
        pltpu.VMEM_SHARED((_N,), _f32),
        pltpu.SemaphoreType.DMA,
        pltpu.SemaphoreType.DMA,
    ],
)
def _deg_kernel(dst_hbm, out_hbm, idx_a, idx_b, idx_t, ones_v, buf_v,
                acc_sh, sem_a, sem_b):
    c = lax.axis_index("c")
    s = lax.axis_index("s")
    wid = c * _NS + s
    e0 = wid * _EPW

    for j in range(_C // 16):
        ones_v[pl.ds(j * 16, 16)] = jnp.ones((16,), _f32)
    for j in range(_DPT // 16):
        buf_v[pl.ds(j * 16, 16)] = jnp.zeros((16,), _f32)
    pltpu.sync_copy(buf_v, acc_sh.at[pl.ds(s * _DPT, _DPT)])

    @pl.when(s == _NS - 1)
    def _():
        pltpu.sync_copy(buf_v.at[pl.ds(0, _DTAIL)],
                        acc_sh.at[pl.ds(_NS * _DPT, _DTAIL)])

    plsc.subcore_barrier()

    def ld(i, idx, sem):
        pltpu.async_copy(dst_hbm.at[pl.ds(e0 + i * _C, _C)], idx, sem)

    def wait(idx, sem):
        pltpu.make_async_copy(dst_hbm.at[pl.ds(e0, _C)], idx, sem).wait()

    ld(0, idx_a, sem_a)

    def body(k, carry):
        i = 2 * k
        ld(i + 1, idx_b, sem_b)
        wait(idx_a, sem_a)
        pltpu.sync_copy(ones_v, acc_sh.at[idx_a], add=True)

        @pl.when(i + 2 < _M)
        def _():
            ld(i + 2, idx_a, sem_a)

        wait(idx_b, sem_b)
        pltpu.sync_copy(ones_v, acc_sh.at[idx_b], add=True)
        return carry

    lax.fori_loop(0, _M // 2, body, 0)
    pltpu.sync_copy(dst_hbm.at[pl.ds(e0 + _M * _C, _CT)], idx_t)
    pltpu.sync_copy(ones_v.at[pl.ds(0, _CT)], acc_sh.at[idx_t], add=True)
    plsc.subcore_barrier()

    pltpu.sync_copy(acc_sh.at[pl.ds(s * _DPT, _DPT)], buf_v)
    pltpu.sync_copy(buf_v, out_hbm.at[pl.ds(c * _N + s * _DPT, _DPT)])

    @pl.when(s == _NS - 1)
    def _():
        pltpu.sync_copy(acc_sh.at[pl.ds(_NS * _DPT, _DTAIL)],
                        buf_v.at[pl.ds(0, _DTAIL)])
        pltpu.sync_copy(buf_v.at[pl.ds(0, _DTAIL)],
                        out_hbm.at[pl.ds(c * _N + _NS * _DPT, _DTAIL)])


@functools.partial(
    pl.kernel,
    out_type=jax.ShapeDtypeStruct((_NC, _N, _D), _f32),
    mesh=_mesh,
    scratch_types=[
        pltpu.VMEM((_C,), jnp.int32),
        pltpu.VMEM((_C,), jnp.int32),
        pltpu.VMEM((_C,), jnp.int32),
        pltpu.VMEM((_C,), jnp.int32),
        pltpu.VMEM((_CT,), jnp.int32),
        pltpu.VMEM((_CT,), jnp.int32),
        pltpu.VMEM((_C, _D), _f32),
        pltpu.VMEM((_C, _D), _f32),
        pltpu.VMEM((_RC, _D), _f32),
        pltpu.VMEM_SHARED((_N, _D), _f32),
        pltpu.SemaphoreType.DMA,
        pltpu.SemaphoreType.DMA,
    ],
)
def _hop_kernel(g_hbm, src_hbm, dst_hbm, out_hbm,
                src_a, src_b, dst_a, dst_b, src_t, dst_t,
                rows_a, rows_b, buf, acc_sh, sem_a, sem_b):
    c = lax.axis_index("c")
    s = lax.axis_index("s")
    wid = c * _NS + s
    e0 = wid * _EPW

    def zrow(r, carry):
        for j in range(_D // 16):
            buf[r, pl.ds(j * 16, 16)] = jnp.zeros((16,), _f32)
        return carry

    lax.fori_loop(0, _RC, zrow, 0)
    for k in range(_RPT // _RC):
        pltpu.sync_copy(buf, acc_sh.at[pl.ds(s * _RPT + k * _RC, _RC)])

    @pl.when(s == _NS - 1)
    def _():
        pltpu.sync_copy(buf.at[pl.ds(0, _RTAIL)],
                        acc_sh.at[pl.ds(_NS * _RPT, _RTAIL)])

    plsc.subcore_barrier()

    def fire(i, srcv, dstv, rows, sem):
        off = e0 + i * _C
        pltpu.sync_copy(src_hbm.at[pl.ds(off, _C)], srcv)
        pltpu.sync_copy(dst_hbm.at[pl.ds(off, _C)], dstv)
        pltpu.async_copy(g_hbm.at[srcv], rows, sem)

    def drain(srcv, rows, sem):
        pltpu.make_async_copy(g_hbm.at[srcv], rows, sem).wait()

    fire(0, src_a, dst_a, rows_a, sem_a)

    def body(k, carry):
        i = 2 * k
        fire(i + 1, src_b, dst_b, rows_b, sem_b)
        drain(src_a, rows_a, sem_a)
        pltpu.sync_copy(rows_a, acc_sh.at[dst_a], add=True)

        @pl.when(i + 2 < _M)
        def _():
            fire(i + 2, src_a, dst_a, rows_a, sem_a)

        drain(src_b, rows_b, sem_b)
        pltpu.sync_copy(rows_b, acc_sh.at[dst_b], add=True)
        return carry

    lax.fori_loop(0, _M // 2, body, 0)

    off_t = e0 + _M * _C
    pltpu.sync_copy(src_hbm.at[pl.ds(off_t, _CT)], src_t)
    pltpu.sync_copy(dst_hbm.at[pl.ds(off_t, _CT)], dst_t)
    pltpu.async_copy(g_hbm.at[src_t], rows_a.at[pl.ds(0, _CT)], sem_a).wait()
    pltpu.sync_copy(rows_a.at[pl.ds(0, _CT)], acc_sh.at[dst_t], add=True)
    plsc.subcore_barrier()

    for k in range(_RPT // _RC):
        r0 = s * _RPT + k * _RC
        pltpu.sync_copy(acc_sh.at[pl.ds(r0, _RC)], buf)
        pltpu.sync_copy(buf, out_hbm.at[c, pl.ds(r0, _RC)])

    @pl.when(s == _NS - 1)
    def _():
        pltpu.sync_copy(acc_sh.at[pl.ds(_NS * _RPT, _RTAIL)],
                        buf.at[pl.ds(0, _RTAIL)])
        pltpu.sync_copy(buf.at[pl.ds(0, _RTAIL)],
                        out_hbm.at[c, pl.ds(_NS * _RPT, _RTAIL)])


_BR = 2000


def _norm_mul_body(degp_ref, feat_ref, norm_ref, g1_ref):
    d = degp_ref[0] + degp_ref[1]
    nv = lax.rsqrt(jnp.maximum(d, 1.0))
    norm_ref[...] = nv
    g1_ref[...] = feat_ref[...] * nv


def _mid_body(p_ref, norm_ref, g2_ref):
    nv = norm_ref[...]
    g2_ref[...] = (p_ref[0] + p_ref[1]) * (nv * nv)


def _fc_body(p_ref, norm_ref, w_ref, b_ref, out_ref):
    h = (p_ref[0] + p_ref[1]) * norm_ref[...]
    out_ref[...] = (
        jnp.dot(h, w_ref[...], preferred_element_type=_f32) + b_ref[...]
    )


_norm_call = pl.pallas_call(
    _norm_mul_body,
    grid=(_N // _BR,),
    in_specs=[
        pl.BlockSpec((_NC, _BR, 1), lambda i: (0, i, 0)),
        pl.BlockSpec((_BR, _D), lambda i: (i, 0)),
    ],
    out_specs=[
        pl.BlockSpec((_BR, 1), lambda i: (i, 0)),
        pl.BlockSpec((_BR, _D), lambda i: (i, 0)),
    ],
    out_shape=[
        jax.ShapeDtypeStruct((_N, 1), _f32),
        jax.ShapeDtypeStruct((_N, _D), _f32),
    ],
)

_mid_call = pl.pallas_call(
    _mid_body,
    grid=(_N // _BR,),
    in_specs=[
        pl.BlockSpec((_NC, _BR, _D), lambda i: (0, i, 0)),
        pl.BlockSpec((_BR, 1), lambda i: (i, 0)),
    ],
    out_specs=pl.BlockSpec((_BR, _D), lambda i: (i, 0)),
    out_shape=jax.ShapeDtypeStruct((_N, _D), _f32),
)

_fc_call = pl.pallas_call(
    _fc_body,
    grid=(_N // _BR,),
    in_specs=[
        pl.BlockSpec((_NC, _BR, _D), lambda i: (0, i, 0)),
        pl.BlockSpec((_BR, 1), lambda i: (i, 0)),
        pl.BlockSpec((_D, _D), lambda i: (0, 0)),
        pl.BlockSpec((1, _D), lambda i: (0, 0)),
    ],
    out_specs=pl.BlockSpec((_BR, _D), lambda i: (i, 0)),
    out_shape=jax.ShapeDtypeStruct((_N, _D), _f32),
)


def kernel(feat, edge_index, W, b):
    src = edge_index[0]
    dst = edge_index[1]
    degp = _deg_kernel(dst)
    norm, g1 = _norm_call(degp.reshape(_NC, _N, 1), feat)
    p1 = _hop_kernel(g1, src, dst)
    g2 = _mid_call(p1, norm)
    p2 = _hop_kernel(g2, src, dst)
    out = _fc_call(p2, norm, W, b.reshape(1, _D))
    return out
